# baseline (speedup 1.0000x reference)
import functools

import numpy as np
import jax
import jax.numpy as jnp
from jax import lax
from jax.experimental import pallas as pl
from jax.experimental.pallas import tpu as pltpu
from jax.experimental.pallas import tpu_sc as plsc

_B = 8
_N = 1024
_K = 20
_EPS = 1e-5
_INV = float(1.0 / np.sqrt(1.0 + _EPS))
_NEG = -1e30
_BF = jnp.bfloat16


def _leaky(x):
    return jnp.where(x >= 0, x, 0.2 * x)


def _bmm(a, b):
    return lax.dot_general(
        a.astype(_BF), b.astype(_BF), (((1,), (1,)), ((), ())),
        preferred_element_type=jnp.float32)


_BPG = 2


def _knn_body(C, x_ref, idx_ref):
    iota = lax.broadcasted_iota(jnp.int32, (_N, _N), 1)
    iota_k = lax.broadcasted_iota(jnp.int32, (_N, _K), 1)

    Ss, accs = [], []
    for bb in range(_BPG):
        feat = x_ref[bb][:, :C]
        inner = -2.0 * _bmm(feat, feat)
        xx = jnp.sum(feat * feat, axis=1, keepdims=True)
        S = (-jnp.transpose(xx) - inner) - xx
        Ss.append(S)
        accs.append(jnp.zeros((_N, _K), jnp.int32))

    def step(t, carry):
        Ss, accs = carry
        nS, nacc = [], []
        for S, acc in zip(Ss, accs):
            m = jnp.max(S, axis=1, keepdims=True)
            ge = S >= m
            j = jnp.min(jnp.where(ge, iota, _N), axis=1, keepdims=True)
            nacc.append(jnp.where(iota_k == t, j, acc))
            nS.append(jnp.where(iota == j, _NEG, S))
        return tuple(nS), tuple(nacc)

    _, accs = lax.fori_loop(0, _K, step, (tuple(Ss), tuple(accs)))
    for bb in range(_BPG):
        idx_ref[bb] = accs[bb] + (pl.program_id(0) * _BPG + bb) * _N


def _knn_call(C, Cpad, nb):
    return pl.pallas_call(
        functools.partial(_knn_body, C),
        grid=(nb // _BPG,),
        in_specs=[pl.BlockSpec((_BPG, _N, Cpad), lambda b: (b, 0, 0))],
        out_specs=pl.BlockSpec((_BPG, _N, _K), lambda b: (b, 0, 0)),
        out_shape=jax.ShapeDtypeStruct((nb, _N, _K), jnp.int32))


def _gather(feat2d, idxf, Cpad):
    BN = feat2d.shape[0]
    E = idxf.shape[0]
    NW = 32
    EPW = E // NW
    RC = 80
    NCH = EPW // RC
    NBUF = 4
    mesh = plsc.VectorSubcoreMesh(core_axis_name="c", subcore_axis_name="s")

    @functools.partial(
        pl.kernel,
        mesh=mesh,
        out_type=jax.ShapeDtypeStruct((E, Cpad), jnp.float32),
        scratch_types=(
            [pltpu.VMEM((EPW,), jnp.int32)]
            + [pltpu.VMEM((RC, Cpad), jnp.float32)] * NBUF
            + [pltpu.SemaphoreType.DMA] * (2 * NBUF)
        ),
    )
    def gk(f_hbm, i_hbm, o_hbm, idx_v, *bufs_sems):
        rows = bufs_sems[:NBUF]
        gsem = bufs_sems[NBUF:2 * NBUF]
        osem = bufs_sems[2 * NBUF:]
        wid = lax.axis_index("s") * 2 + lax.axis_index("c")
        base = wid * EPW
        pltpu.sync_copy(i_hbm.at[pl.ds(base, EPW)], idx_v)

        def sgather(g, b):
            pltpu.async_copy(
                f_hbm.at[idx_v.at[pl.ds(g * RC, RC)]], rows[b], gsem[b])

        def wgather(g, b):
            pltpu.make_async_copy(
                f_hbm.at[idx_v.at[pl.ds(g * RC, RC)]], rows[b],
                gsem[b]).wait()

        def swrite(g, b):
            pltpu.async_copy(rows[b], o_hbm.at[pl.ds(base + g * RC, RC)],
                             osem[b])

        def wwrite(g, b):
            pltpu.make_async_copy(rows[b],
                                  o_hbm.at[pl.ds(base + g * RC, RC)],
                                  osem[b]).wait()

        for b in range(NBUF):
            sgather(b, b)

        @pl.loop(0, NCH // NBUF)
        def _(i):
            for b in range(NBUF):
                g = i * NBUF + b
                wgather(g, b)
                swrite(g, b)
                wwrite(g, b)
                nxt = g + NBUF

                @pl.when(nxt < NCH)
                def _():
                    sgather(nxt, b)

    return gk(feat2d, idxf)


def _conv_body(C, O, Opad, e_ref, x_ref, W_ref, g_ref, b_ref, y_ref):
    feat = x_ref[0][:, :C]
    e3 = e_ref[0].reshape(_N, _K, -1)[:, :, :C]
    diff = (e3 - feat[:, None, :]).reshape(_N * _K, C)
    yd = _bmm(diff, W_ref[:, :C]).reshape(_N, _K, O)
    md = jnp.max(yd, axis=1)
    pt = _bmm(feat, W_ref[:, C:])
    y = (md + pt) * (g_ref[...] * _INV) + b_ref[...]
    y = _leaky(y)
    if Opad > O:
        y = jnp.concatenate(
            [y, jnp.zeros((_N, Opad - O), jnp.float32)], axis=1)
    y_ref[0] = y


def _conv_call(C, Cpad, O, Opad, nb):
    return pl.pallas_call(
        functools.partial(_conv_body, C, O, Opad),
        grid=(nb,),
        in_specs=[
            pl.BlockSpec((1, _N * _K, Cpad), lambda b: (b, 0, 0)),
            pl.BlockSpec((1, _N, Cpad), lambda b: (b, 0, 0)),
            pl.BlockSpec((O, 2 * C), lambda b: (0, 0)),
            pl.BlockSpec((1, O), lambda b: (0, 0)),
            pl.BlockSpec((1, O), lambda b: (0, 0)),
        ],
        out_specs=pl.BlockSpec((1, _N, Opad), lambda b: (b, 0, 0)),
        out_shape=jax.ShapeDtypeStruct((nb, _N, Opad), jnp.float32))


def _final_body(*refs):
    (y1, y2, y3, y4, W5, g5, b5, Wf1, gf1, bf1, Wf2, gf2, bf2, Wf3, bf3,
     out_ref, p1_ref) = refs

    cat = jnp.concatenate(
        [y1[0][:, :64], y2[0][:, :64], y3[0], y4[0]], axis=1)
    h = _bmm(cat, W5[...])
    h = _leaky(h * (g5[...] * _INV) + b5[...])
    p1 = jnp.max(h, axis=0, keepdims=True)
    p2 = jnp.sum(h, axis=0, keepdims=True) * (1.0 / _N)
    f = jnp.concatenate([p1, p2], axis=1)
    f = _leaky(_bmm(f, Wf1[...]) * (gf1[...] * _INV) + bf1[...])
    f = _leaky(_bmm(f, Wf2[...]) * (gf2[...] * _INV) + bf2[...])
    logits = _bmm(f, Wf3[...]) + bf3[...]
    z = logits - jnp.max(logits, axis=1, keepdims=True)
    out_ref[0] = z - jnp.log(jnp.sum(jnp.exp(z), axis=1, keepdims=True))
    p1_ref[0] = p1


def _final_call(n_classes):
    def row(c):
        return pl.BlockSpec((1, c), lambda b: (0, 0))

    def bnc(c):
        return pl.BlockSpec((1, _N, c), lambda b: (b, 0, 0))

    def w(o, c):
        return pl.BlockSpec((o, c), lambda b: (0, 0))

    in_specs = [
        bnc(128), bnc(128), bnc(128), bnc(256),
        w(1024, 512), row(1024), row(1024),
        w(512, 2048), row(512), row(512),
        w(256, 512), row(256), row(256),
        w(n_classes, 256), row(n_classes),
    ]
    out_specs = [
        pl.BlockSpec((1, 1, n_classes), lambda b: (b, 0, 0)),
        pl.BlockSpec((1, 1, _N), lambda b: (b, 0, 0)),
    ]
    out_shape = [
        jax.ShapeDtypeStruct((_B, 1, n_classes), jnp.float32),
        jax.ShapeDtypeStruct((_B, 1, _N), jnp.float32),
    ]
    return pl.pallas_call(_final_body, grid=(_B,), in_specs=in_specs,
                          out_specs=out_specs, out_shape=out_shape)


def _stage(feats, C, Cpad, W, g, b, O, Opad):
    nb = feats.shape[0]
    idx = _knn_call(C, Cpad, nb)(feats)
    e = _gather(feats.reshape(-1, Cpad), idx.reshape(-1), Cpad)
    return _conv_call(C, Cpad, O, Opad, nb)(
        e.reshape(nb, _N * _K, Cpad), feats, W,
        g.reshape(1, -1), b.reshape(1, -1))


def kernel(x, W1, g1, b1, W2, g2, b2, W3, g3, b3, W4, g4, b4, W5, g5, b5,
           Wf1, gf1, bf1, Wf2, gf2, bf2, Wf3, bf3):
    xT = jnp.transpose(x, (0, 2, 1))
    x0 = jnp.pad(xT, ((0, 0), (0, 0), (0, 125)))

    H = _B // 2
    ys = []
    for lo in (0, H):
        f = x0[lo:lo + H]
        a1 = _stage(f, 3, 128, W1, g1, b1, 64, 128)
        a2 = _stage(a1, 64, 128, W2, g2, b2, 64, 128)
        a3 = _stage(a2, 64, 128, W3, g3, b3, 128, 128)
        a4 = _stage(a3, 128, 128, W4, g4, b4, 256, 256)
        ys.append((a1, a2, a3, a4))
    y1, y2, y3, y4 = (jnp.concatenate([ys[0][i], ys[1][i]], axis=0)
                      for i in range(4))

    n_classes = Wf3.shape[0]
    out, p1 = _final_call(n_classes)(
        y1, y2, y3, y4,
        W5, g5.reshape(1, -1), b5.reshape(1, -1),
        Wf1, gf1.reshape(1, -1), bf1.reshape(1, -1),
        Wf2, gf2.reshape(1, -1), bf2.reshape(1, -1),
        Wf3, bf3.reshape(1, -1))
    return (out.reshape(_B, n_classes), p1.reshape(_B, _N))


# speedup vs baseline: 1.0008x; 1.0008x over previous
import functools

import numpy as np
import jax
import jax.numpy as jnp
from jax import lax
from jax.experimental import pallas as pl
from jax.experimental.pallas import tpu as pltpu
from jax.experimental.pallas import tpu_sc as plsc

_B = 8
_N = 1024
_K = 20
_EPS = 1e-5
_INV = float(1.0 / np.sqrt(1.0 + _EPS))
_NEG = -1e30
_BF = jnp.bfloat16


def _leaky(x):
    return jnp.where(x >= 0, x, 0.2 * x)


def _bmm(a, b):
    return lax.dot_general(
        a.astype(_BF), b.astype(_BF), (((1,), (1,)), ((), ())),
        preferred_element_type=jnp.float32)


_BPG = 2


def _knn_body(C, x_ref, idx_ref):
    iota = lax.broadcasted_iota(jnp.int32, (_N, _N), 1)
    iota_k = lax.broadcasted_iota(jnp.int32, (_N, _K), 1)

    Ss, accs = [], []
    for bb in range(_BPG):
        feat = x_ref[bb][:, :C]
        inner = -2.0 * _bmm(feat, feat)
        xx = jnp.sum(feat * feat, axis=1, keepdims=True)
        S = (-jnp.transpose(xx) - inner) - xx
        Ss.append(S)
        accs.append(jnp.zeros((_N, _K), jnp.int32))

    def step(t, carry):
        Ss, accs = carry
        nS, nacc = [], []
        for S, acc in zip(Ss, accs):
            m = jnp.max(S, axis=1, keepdims=True)
            ge = S >= m
            j = jnp.min(jnp.where(ge, iota, _N), axis=1, keepdims=True)
            nacc.append(jnp.where(iota_k == t, j, acc))
            nS.append(jnp.where(iota == j, _NEG, S))
        return tuple(nS), tuple(nacc)

    _, accs = lax.fori_loop(0, _K, step, (tuple(Ss), tuple(accs)))
    for bb in range(_BPG):
        idx_ref[bb] = accs[bb] + (pl.program_id(0) * _BPG + bb) * _N


def _knn_call(C, Cpad, nb):
    return pl.pallas_call(
        functools.partial(_knn_body, C),
        grid=(nb // _BPG,),
        in_specs=[pl.BlockSpec((_BPG, _N, Cpad), lambda b: (b, 0, 0))],
        out_specs=pl.BlockSpec((_BPG, _N, _K), lambda b: (b, 0, 0)),
        out_shape=jax.ShapeDtypeStruct((nb, _N, _K), jnp.int32))


def _gather(feat2d, idxf, Cpad):
    BN = feat2d.shape[0]
    E = idxf.shape[0]
    NW = 32
    EPW = E // NW
    RC = 80
    NCH = EPW // RC
    NBUF = 4
    mesh = plsc.VectorSubcoreMesh(core_axis_name="c", subcore_axis_name="s")

    @functools.partial(
        pl.kernel,
        mesh=mesh,
        out_type=jax.ShapeDtypeStruct((E, Cpad), jnp.float32),
        scratch_types=(
            [pltpu.VMEM((EPW,), jnp.int32)]
            + [pltpu.VMEM((RC, Cpad), jnp.float32)] * NBUF
            + [pltpu.SemaphoreType.DMA] * (2 * NBUF)
        ),
    )
    def gk(f_hbm, i_hbm, o_hbm, idx_v, *bufs_sems):
        rows = bufs_sems[:NBUF]
        gsem = bufs_sems[NBUF:2 * NBUF]
        osem = bufs_sems[2 * NBUF:]
        wid = lax.axis_index("s") * 2 + lax.axis_index("c")
        base = wid * EPW
        pltpu.sync_copy(i_hbm.at[pl.ds(base, EPW)], idx_v)

        def sgather(g, b):
            pltpu.async_copy(
                f_hbm.at[idx_v.at[pl.ds(g * RC, RC)]], rows[b], gsem[b])

        def wgather(g, b):
            pltpu.make_async_copy(
                f_hbm.at[idx_v.at[pl.ds(g * RC, RC)]], rows[b],
                gsem[b]).wait()

        def swrite(g, b):
            pltpu.async_copy(rows[b], o_hbm.at[pl.ds(base + g * RC, RC)],
                             osem[b])

        def wwrite(g, b):
            pltpu.make_async_copy(rows[b],
                                  o_hbm.at[pl.ds(base + g * RC, RC)],
                                  osem[b]).wait()

        for b in range(NBUF):
            sgather(b, b)

        @pl.loop(0, NCH // NBUF)
        def _(i):
            for b in range(NBUF):
                g = i * NBUF + b
                wgather(g, b)
                swrite(g, b)
                wwrite(g, b)
                nxt = g + NBUF

                @pl.when(nxt < NCH)
                def _():
                    sgather(nxt, b)

    return gk(feat2d, idxf)


def _conv_body(C, O, Opad, e_ref, x_ref, W_ref, g_ref, b_ref, y_ref):
    feat = x_ref[0][:, :C]
    e3 = e_ref[0].reshape(_N, _K, -1)[:, :, :C]
    diff = (e3 - feat[:, None, :]).reshape(_N * _K, C)
    yd = _bmm(diff, W_ref[:, :C]).reshape(_N, _K, O)
    md = jnp.max(yd, axis=1)
    pt = _bmm(feat, W_ref[:, C:])
    y = (md + pt) * (g_ref[...] * _INV) + b_ref[...]
    y = _leaky(y)
    if Opad > O:
        y = jnp.concatenate(
            [y, jnp.zeros((_N, Opad - O), jnp.float32)], axis=1)
    y_ref[0] = y


def _conv_call(C, Cpad, O, Opad, nb):
    return pl.pallas_call(
        functools.partial(_conv_body, C, O, Opad),
        grid=(nb,),
        in_specs=[
            pl.BlockSpec((1, _N * _K, Cpad), lambda b: (b, 0, 0)),
            pl.BlockSpec((1, _N, Cpad), lambda b: (b, 0, 0)),
            pl.BlockSpec((O, 2 * C), lambda b: (0, 0)),
            pl.BlockSpec((1, O), lambda b: (0, 0)),
            pl.BlockSpec((1, O), lambda b: (0, 0)),
        ],
        out_specs=pl.BlockSpec((1, _N, Opad), lambda b: (b, 0, 0)),
        out_shape=jax.ShapeDtypeStruct((nb, _N, Opad), jnp.float32))


def _final_body(*refs):
    (y1, y2, y3, y4, W5, g5, b5, Wf1, gf1, bf1, Wf2, gf2, bf2, Wf3, bf3,
     out_ref, p1_ref) = refs

    cat = jnp.concatenate(
        [y1[0][:, :64], y2[0][:, :64], y3[0], y4[0]], axis=1)
    h = _bmm(cat, W5[...])
    h = _leaky(h * (g5[...] * _INV) + b5[...])
    p1 = jnp.max(h, axis=0, keepdims=True)
    p2 = jnp.sum(h, axis=0, keepdims=True) * (1.0 / _N)
    f = jnp.concatenate([p1, p2], axis=1)
    f = _leaky(_bmm(f, Wf1[...]) * (gf1[...] * _INV) + bf1[...])
    f = _leaky(_bmm(f, Wf2[...]) * (gf2[...] * _INV) + bf2[...])
    logits = _bmm(f, Wf3[...]) + bf3[...]
    z = logits - jnp.max(logits, axis=1, keepdims=True)
    out_ref[0] = z - jnp.log(jnp.sum(jnp.exp(z), axis=1, keepdims=True))
    p1_ref[0] = p1


def _final_call(n_classes):
    def row(c):
        return pl.BlockSpec((1, c), lambda b: (0, 0))

    def bnc(c):
        return pl.BlockSpec((1, _N, c), lambda b: (b, 0, 0))

    def w(o, c):
        return pl.BlockSpec((o, c), lambda b: (0, 0))

    in_specs = [
        bnc(128), bnc(128), bnc(128), bnc(256),
        w(1024, 512), row(1024), row(1024),
        w(512, 2048), row(512), row(512),
        w(256, 512), row(256), row(256),
        w(n_classes, 256), row(n_classes),
    ]
    out_specs = [
        pl.BlockSpec((1, 1, n_classes), lambda b: (b, 0, 0)),
        pl.BlockSpec((1, 1, _N), lambda b: (b, 0, 0)),
    ]
    out_shape = [
        jax.ShapeDtypeStruct((_B, 1, n_classes), jnp.float32),
        jax.ShapeDtypeStruct((_B, 1, _N), jnp.float32),
    ]
    return pl.pallas_call(_final_body, grid=(_B,), in_specs=in_specs,
                          out_specs=out_specs, out_shape=out_shape)


def _stage(feats, C, Cpad, W, g, b, O, Opad):
    nb = feats.shape[0]
    idx = _knn_call(C, Cpad, nb)(feats)
    e = _gather(feats.reshape(-1, Cpad), idx.reshape(-1), Cpad)
    return _conv_call(C, Cpad, O, Opad, nb)(
        e.reshape(nb, _N * _K, Cpad), feats, W,
        g.reshape(1, -1), b.reshape(1, -1))


def kernel(x, W1, g1, b1, W2, g2, b2, W3, g3, b3, W4, g4, b4, W5, g5, b5,
           Wf1, gf1, bf1, Wf2, gf2, bf2, Wf3, bf3):
    xT = jnp.transpose(x, (0, 2, 1))
    x0 = jnp.pad(xT, ((0, 0), (0, 0), (0, 125)))

    H = _B // 2
    params = [(3, W1, g1, b1, 64), (64, W2, g2, b2, 64),
              (64, W3, g3, b3, 128), (128, W4, g4, b4, 256)]
    fa, fb = x0[:H], x0[H:]
    ys = []
    for si, (C, W, g, b, O) in enumerate(params):
        Opad = 256 if O == 256 else 128
        fa = _stage(fa, C, 128, W, g, b, O, Opad)
        fb = _stage(fb, C, 128, W, g, b, O, Opad)
        ys.append(jnp.concatenate([fa, fb], axis=0))
    y1, y2, y3, y4 = ys

    n_classes = Wf3.shape[0]
    out, p1 = _final_call(n_classes)(
        y1, y2, y3, y4,
        W5, g5.reshape(1, -1), b5.reshape(1, -1),
        Wf1, gf1.reshape(1, -1), bf1.reshape(1, -1),
        Wf2, gf2.reshape(1, -1), bf2.reshape(1, -1),
        Wf3, bf3.reshape(1, -1))
    return (out.reshape(_B, n_classes), p1.reshape(_B, _N))
